# baseline (speedup 1.0000x reference)
import functools

import jax
import jax.numpy as jnp
import numpy as np
from jax import lax
from jax.experimental import pallas as pl
from jax.experimental.pallas import tpu as pltpu
from jax.experimental.pallas import tpu_sc as plsc

N = 10000
D = 128
NR = 16
NREG = 5000
B = 1024
EDGES = 320000
MPN = 8000
MPE = 256000

NC = 2
NS = 16
NW = NC * NS
CH = 128

CPW1 = 80
EP1 = NW * CPW1 * CH
ACC1 = 10240
PAD1 = EP1 - EDGES

CPW2 = 128
EP2 = NS * CPW2 * CH
PAD2 = EP2 - MPE
ACC2 = 8192
FROWS = 5120
FCH = FROWS // CH

SEG1 = 2
SEG2 = 2


def _zero_vmem_2d(ref, nrows):
    def row(r, _):
        for j in range(D // 16):
            ref[r, pl.ds(16 * j, 16)] = jnp.zeros((16,), jnp.float32)
        return 0
    lax.fori_loop(0, nrows, row, 0)


def _fill_vmem_1d(ref, n, val):
    for j in range(n // 16):
        ref[pl.ds(16 * j, 16)] = jnp.full((16,), val, jnp.float32)


def _sc_mesh():
    return plsc.VectorSubcoreMesh(core_axis_name="c", subcore_axis_name="s")


def _sc1_body(src_hbm, dst_hbm, table_hbm, acc_out, cnt_out0, cnt_out1,
              idx_s, idx_d, rows0, rows1, ones, zvec, acc_sh, cnt_sh,
              gs0, gs1, ss0, ss1, cs0):
    c = lax.axis_index("c")
    s = lax.axis_index("s")
    w = c * NS + s

    _zero_vmem_2d(rows0, CH)
    _fill_vmem_1d(zvec, ACC1 // NS, 0.0)
    _fill_vmem_1d(ones, CH, 1.0)

    rpt = ACC1 // NS
    for k in range(rpt // CH):
        pltpu.sync_copy(rows0, acc_sh.at[pl.ds(s * rpt + k * CH, CH)])
    pltpu.sync_copy(zvec, cnt_sh.at[pl.ds(s * rpt, rpt)])
    plsc.subcore_barrier()

    def drain_s(rbuf, sem_s):
        pltpu.make_async_copy(rbuf, acc_sh.at[idx_d.at[0]], sem_s).wait()

    seg = CPW1 // SEG1
    for sg in range(SEG1):
        base = w * CPW1 + sg * seg
        pltpu.sync_copy(src_hbm.at[pl.ds(base, seg)], idx_s)
        pltpu.sync_copy(dst_hbm.at[pl.ds(base, seg)], idx_d)

        def rnd(r, _):
            @pl.when(r > 0)
            def _():
                drain_s(rows0, ss0)
                drain_s(rows1, ss1)
            g0 = pltpu.async_copy(table_hbm.at[idx_s.at[2 * r]], rows0, gs0)
            g1 = pltpu.async_copy(table_hbm.at[idx_s.at[2 * r + 1]], rows1,
                                  gs1)
            g0.wait()
            pltpu.async_copy(rows0, acc_sh.at[idx_d.at[2 * r]], ss0,
                             add=True)
            g1.wait()
            pltpu.async_copy(rows1, acc_sh.at[idx_d.at[2 * r + 1]], ss1,
                             add=True)
            return 0
        lax.fori_loop(0, seg // 2, rnd, 0)
        drain_s(rows0, ss0)
        drain_s(rows1, ss1)

        cds = [pltpu.async_copy(ones, cnt_sh.at[idx_d.at[j]], cs0, add=True)
               for j in range(seg)]
        for dsc in cds:
            dsc.wait()

    plsc.subcore_barrier()
    pltpu.sync_copy(acc_sh.at[pl.ds(s * rpt, rpt)],
                    acc_out.at[c, pl.ds(s * rpt, rpt)])

    pltpu.sync_copy(cnt_sh.at[pl.ds(s * rpt, rpt)], zvec)

    @pl.when(c == 0)
    def _():
        pltpu.sync_copy(zvec, cnt_out0.at[pl.ds(s * rpt, rpt)])

    @pl.when(c == 1)
    def _():
        pltpu.sync_copy(zvec, cnt_out1.at[pl.ds(s * rpt, rpt)])


def _run_sc1(src2d, dst2d, table):
    f = pl.kernel(
        _sc1_body,
        out_type=(jax.ShapeDtypeStruct((NC, ACC1, D), jnp.float32),
                  jax.ShapeDtypeStruct((ACC1,), jnp.float32),
                  jax.ShapeDtypeStruct((ACC1,), jnp.float32)),
        mesh=_sc_mesh(),
        scratch_types=[
            pltpu.VMEM((CPW1 // SEG1, CH), jnp.int32),
            pltpu.VMEM((CPW1 // SEG1, CH), jnp.int32),
            pltpu.VMEM((CH, D), jnp.float32),
            pltpu.VMEM((CH, D), jnp.float32),
            pltpu.VMEM((CH,), jnp.float32),
            pltpu.VMEM((ACC1 // NS,), jnp.float32),
            pltpu.VMEM_SHARED((ACC1, D), jnp.float32),
            pltpu.VMEM_SHARED((ACC1,), jnp.float32),
            pltpu.SemaphoreType.DMA,
            pltpu.SemaphoreType.DMA,
            pltpu.SemaphoreType.DMA,
            pltpu.SemaphoreType.DMA,
            pltpu.SemaphoreType.DMA,
        ],
    )
    return f(src2d, dst2d, table)


def _sc2_body(e_hbm, rw_hbm, src0, dst0, src1, dst1, eids0, eids1,
              fidx0, fidx1, hidx, ridx,
              mp_acc, mp_cnt0, mp_cnt1, feat, hg, rg,
              idx_s, idx_d, rows0, rows1, ones, zvec, cidx,
              fidx_v, hridx_v, acc_sh, cnt_sh,
              es0, gs0, gs1, ss0, ss1, cs0):
    c = lax.axis_index("c")
    s = lax.axis_index("s")

    _zero_vmem_2d(rows0, CH)
    _fill_vmem_1d(zvec, ACC2 // NS, 0.0)
    _fill_vmem_1d(ones, CH, 1.0)

    rpt = ACC2 // NS
    for k in range(rpt // CH):
        pltpu.sync_copy(rows0, acc_sh.at[pl.ds(s * rpt + k * CH, CH)])
    pltpu.sync_copy(zvec, cnt_sh.at[pl.ds(s * rpt, rpt)])
    plsc.subcore_barrier()

    def drain_s(rbuf, sem_s):
        pltpu.make_async_copy(rbuf, acc_sh.at[idx_d.at[0]], sem_s).wait()

    def graph(src_hbm, dst_hbm, eids_hbm):
        seg = CPW2 // SEG2
        for sg in range(SEG2):
            base = s * CPW2 + sg * seg
            pltpu.sync_copy(src_hbm.at[pl.ds(base, seg)], idx_s)
            pltpu.sync_copy(dst_hbm.at[pl.ds(base, seg)], idx_d)

            comp = [pltpu.async_copy(eids_hbm.at[idx_s.at[j]], cidx.at[j],
                                     es0) for j in range(seg)]
            for dsc in comp:
                dsc.wait()

            def rnd(r, _):
                @pl.when(r > 0)
                def _():
                    drain_s(rows0, ss0)
                    drain_s(rows1, ss1)
                g0 = pltpu.async_copy(e_hbm.at[cidx.at[2 * r]], rows0, gs0)
                g1 = pltpu.async_copy(e_hbm.at[cidx.at[2 * r + 1]], rows1,
                                      gs1)
                g0.wait()
                pltpu.async_copy(rows0, acc_sh.at[idx_d.at[2 * r]], ss0,
                                 add=True)
                g1.wait()
                pltpu.async_copy(rows1, acc_sh.at[idx_d.at[2 * r + 1]], ss1,
                                 add=True)
                return 0
            lax.fori_loop(0, seg // 2, rnd, 0)
            drain_s(rows0, ss0)
            drain_s(rows1, ss1)

            cds = [pltpu.async_copy(ones, cnt_sh.at[idx_d.at[j]], cs0,
                                    add=True) for j in range(seg)]
            for dsc in cds:
                dsc.wait()

    @pl.when(c == 0)
    def _():
        graph(src0, dst0, eids0)

    @pl.when(c == 1)
    def _():
        graph(src1, dst1, eids1)

    @pl.when(c == 0)
    def _():
        pltpu.sync_copy(fidx0, fidx_v)

    @pl.when(c == 1)
    def _():
        pltpu.sync_copy(fidx1, fidx_v)

    for k in range(-(-FCH // NS)):
        ch = s + k * NS

        @pl.when(ch < FCH)
        def _():
            pltpu.sync_copy(e_hbm.at[fidx_v.at[ch]], rows0)
            pltpu.sync_copy(rows0, feat.at[c, pl.ds(ch * CH, CH)])

    @pl.when((c == 0) & (s < B // CH))
    def _():
        pltpu.sync_copy(hidx, hridx_v)
        pltpu.sync_copy(e_hbm.at[hridx_v.at[s]], rows1)
        pltpu.sync_copy(rows1, hg.at[pl.ds(s * CH, CH)])

    @pl.when((c == 1) & (s < B // CH))
    def _():
        pltpu.sync_copy(ridx, hridx_v)
        pltpu.sync_copy(rw_hbm.at[hridx_v.at[s]], rows1)
        pltpu.sync_copy(rows1, rg.at[pl.ds(s * CH, CH)])

    plsc.subcore_barrier()
    opt = FROWS // NS
    pltpu.sync_copy(acc_sh.at[pl.ds(s * opt, opt)],
                    mp_acc.at[c, pl.ds(s * opt, opt)])

    pltpu.sync_copy(cnt_sh.at[pl.ds(s * opt, opt)], zvec.at[pl.ds(0, opt)])

    @pl.when(c == 0)
    def _():
        pltpu.sync_copy(zvec.at[pl.ds(0, opt)], mp_cnt0.at[pl.ds(s * opt, opt)])

    @pl.when(c == 1)
    def _():
        pltpu.sync_copy(zvec.at[pl.ds(0, opt)], mp_cnt1.at[pl.ds(s * opt, opt)])


def _run_sc2(e, rw, src0, dst0, src1, dst1, eids0, eids1, fidx0, fidx1,
             hidx, ridx):
    f = pl.kernel(
        _sc2_body,
        out_type=(jax.ShapeDtypeStruct((NC, FROWS, D), jnp.float32),
                  jax.ShapeDtypeStruct((FROWS,), jnp.float32),
                  jax.ShapeDtypeStruct((FROWS,), jnp.float32),
                  jax.ShapeDtypeStruct((NC, FROWS, D), jnp.float32),
                  jax.ShapeDtypeStruct((B, D), jnp.float32),
                  jax.ShapeDtypeStruct((B, D), jnp.float32)),
        mesh=_sc_mesh(),
        scratch_types=[
            pltpu.VMEM((CPW2 // SEG2, CH), jnp.int32),
            pltpu.VMEM((CPW2 // SEG2, CH), jnp.int32),
            pltpu.VMEM((CH, D), jnp.float32),
            pltpu.VMEM((CH, D), jnp.float32),
            pltpu.VMEM((CH,), jnp.float32),
            pltpu.VMEM((ACC2 // NS,), jnp.float32),
            pltpu.VMEM((CPW2 // SEG2, CH), jnp.int32),
            pltpu.VMEM((FCH, CH), jnp.int32),
            pltpu.VMEM((B // CH, CH), jnp.int32),
            pltpu.VMEM_SHARED((ACC2, D), jnp.float32),
            pltpu.VMEM_SHARED((ACC2,), jnp.float32),
            pltpu.SemaphoreType.DMA,
            pltpu.SemaphoreType.DMA,
            pltpu.SemaphoreType.DMA,
            pltpu.SemaphoreType.DMA,
            pltpu.SemaphoreType.DMA,
            pltpu.SemaphoreType.DMA,
        ],
    )
    return f(e, rw, src0, dst0, src1, dst1, eids0, eids1, fidx0, fidx1,
             hidx, ridx)


def _tca_body(acc_ref, cnt0_ref, cnt1_ref, ew_ref, w_ref, root_ref, b_ref,
              e_ref):
    acc = acc_ref[0] + acc_ref[1]
    cnt = jnp.maximum(cnt0_ref[...] + cnt1_ref[...], 1.0)
    agg = acc / cnt
    e_ref[...] = jnp.tanh(
        jnp.dot(agg, w_ref[...], preferred_element_type=jnp.float32)
        + jnp.dot(ew_ref[...], root_ref[...], preferred_element_type=jnp.float32)
        + b_ref[...])


def _run_tca(acc_p, cnt0, cnt1, ew, w, root, b):
    blk = 1000
    grid = N // blk
    return pl.pallas_call(
        _tca_body,
        grid=(grid,),
        in_specs=[
            pl.BlockSpec((NC, blk, D), lambda i: (0, i, 0)),
            pl.BlockSpec((blk, 1), lambda i: (i, 0)),
            pl.BlockSpec((blk, 1), lambda i: (i, 0)),
            pl.BlockSpec((blk, D), lambda i: (i, 0)),
            pl.BlockSpec((D, D), lambda i: (0, 0)),
            pl.BlockSpec((D, D), lambda i: (0, 0)),
            pl.BlockSpec((1, D), lambda i: (0, 0)),
        ],
        out_specs=pl.BlockSpec((blk, D), lambda i: (i, 0)),
        out_shape=jax.ShapeDtypeStruct((N, D), jnp.float32),
    )(acc_p, cnt0.reshape(ACC1, 1), cnt1.reshape(ACC1, 1), ew, w, root,
      b.reshape(1, D))


def _tcb_body(h_ref, r_ref, e_ref, out_ref):
    x = h_ref[...] * r_ref[...]
    out_ref[...] = lax.dot_general(
        x, e_ref[...], (((1,), (1,)), ((), ())),
        preferred_element_type=jnp.float32)


def _run_tcb(hg, rg, e):
    blk = 128
    grid = B // blk
    return pl.pallas_call(
        _tcb_body,
        grid=(grid,),
        in_specs=[
            pl.BlockSpec((blk, D), lambda i: (i, 0)),
            pl.BlockSpec((blk, D), lambda i: (i, 0)),
            pl.BlockSpec((N, D), lambda i: (0, 0)),
        ],
        out_specs=pl.BlockSpec((blk, N), lambda i: (i, 0)),
        out_shape=jax.ShapeDtypeStruct((B, N), jnp.float32),
    )(hg, rg, e)


def _tcc_body(acc_ref, cnt0_ref, cnt1_ref, feat_ref, eh_ref,
              w0_ref, r0_ref, b0_ref, w1_ref, r1_ref, b1_ref,
              sw1_ref, sb1_ref, sw2_ref, pw_ref, pb_ref, out_ref):
    gw = (w0_ref, w1_ref)
    gr = (r0_ref, r1_ref)
    gb = (b0_ref, b1_ref)
    gc = (cnt0_ref, cnt1_ref)
    sems = []
    ws = []
    for g in range(2):
        cnt = jnp.maximum(gc[g][:NREG], 1.0)
        agg = acc_ref[g, :NREG] / cnt
        sg = jnp.dot(agg, gw[g][...], preferred_element_type=jnp.float32)
        sg = sg + jnp.dot(feat_ref[g, :NREG], gr[g][...],
                          preferred_element_type=jnp.float32)
        sg = jnp.maximum(sg + gb[g][...], 0.0)
        t = jnp.tanh(jnp.dot(sg, sw1_ref[...],
                             preferred_element_type=jnp.float32) + sb1_ref[...])
        ws.append(jnp.mean(jnp.sum(t * sw2_ref[...], axis=1)))
        sems.append(sg)
    m = jnp.maximum(ws[0], ws[1])
    e0 = jnp.exp(ws[0] - m)
    e1 = jnp.exp(ws[1] - m)
    h_out = (e0 * sems[0] + e1 * sems[1]) / (e0 + e1)
    out_ref[...] = (jnp.dot(h_out, pw_ref[...], preferred_element_type=jnp.float32)
                    + pb_ref[...] + eh_ref[...])


def _run_tcc(mp_acc, mp_cnt0, mp_cnt1, feat, eh, w0, r0, b0, w1, r1, b1,
             sw1, sb1, sw2, pw, pb):
    return pl.pallas_call(
        _tcc_body,
        out_shape=jax.ShapeDtypeStruct((NREG, D), jnp.float32),
    )(mp_acc, mp_cnt0.reshape(FROWS, 1), mp_cnt1.reshape(FROWS, 1), feat,
      eh, w0, r0, b0.reshape(1, D), w1, r1, b1.reshape(1, D), sw1,
      sb1.reshape(1, D), sw2.reshape(1, D), pw, pb.reshape(1, D))


def kernel(E_weight, R_weight, rgcn_W, rgcn_root, rgcn_b,
           gnn0_W, gnn0_root, gnn0_b, gnn1_W, gnn1_root, gnn1_b,
           sem_W1, sem_b1, sem_W2, pred_W, pred_b,
           h_idx, r_idx, edge_index, g0_edge_index, g0_eids,
           g1_edge_index, g1_eids):
    h_idx = h_idx.astype(jnp.int32)
    r_idx = r_idx.astype(jnp.int32)

    pad_src = (jnp.arange(PAD1, dtype=jnp.int32) * 37) % N
    pad_dst = N + (jnp.arange(PAD1, dtype=jnp.int32) % (ACC1 - N))
    src2d = jnp.concatenate([edge_index[0], pad_src]).reshape(EP1 // CH, CH)
    dst2d = jnp.concatenate([edge_index[1], pad_dst]).reshape(EP1 // CH, CH)

    acc_p, cnt0, cnt1 = _run_sc1(src2d, dst2d, E_weight)
    e = _run_tca(acc_p, cnt0, cnt1, E_weight, rgcn_W, rgcn_root, rgcn_b)

    mp_pad_src = (jnp.arange(PAD2, dtype=jnp.int32) * 29) % MPN
    mp_pad_dst = MPN + (jnp.arange(PAD2, dtype=jnp.int32) % (ACC2 - MPN))
    s0 = jnp.concatenate([g0_edge_index[0], mp_pad_src]).reshape(EP2 // CH, CH)
    d0 = jnp.concatenate([g0_edge_index[1], mp_pad_dst]).reshape(EP2 // CH, CH)
    s1 = jnp.concatenate([g1_edge_index[0], mp_pad_src]).reshape(EP2 // CH, CH)
    d1 = jnp.concatenate([g1_edge_index[1], mp_pad_dst]).reshape(EP2 // CH, CH)
    fpad = jnp.zeros((FROWS - NREG,), jnp.int32)
    fidx0 = jnp.concatenate([g0_eids[:NREG], fpad]).reshape(FCH, CH)
    fidx1 = jnp.concatenate([g1_eids[:NREG], fpad]).reshape(FCH, CH)
    hidx = h_idx.reshape(B // CH, CH)
    ridx = r_idx.reshape(B // CH, CH)

    mp_acc, mp_cnt0, mp_cnt1, feat, hg, rg = _run_sc2(
        e, R_weight, s0, d0, s1, d1, g0_eids, g1_eids, fidx0, fidx1,
        hidx, ridx)

    pred = _run_tcb(hg, rg, e)
    e_reg = _run_tcc(mp_acc, mp_cnt0, mp_cnt1, feat, e[:NREG], gnn0_W,
                     gnn0_root, gnn0_b, gnn1_W, gnn1_root, gnn1_b, sem_W1,
                     sem_b1, sem_W2, pred_W, pred_b)
    return (e_reg, pred)


# speedup vs baseline: 1.0618x; 1.0618x over previous
import functools

import jax
import jax.numpy as jnp
import numpy as np
from jax import lax
from jax.experimental import pallas as pl
from jax.experimental.pallas import tpu as pltpu
from jax.experimental.pallas import tpu_sc as plsc

N = 10000
D = 128
NR = 16
NREG = 5000
B = 1024
EDGES = 320000
MPN = 8000
MPE = 256000

NC = 2
NS = 16
NW = NC * NS
CH = 128

CPW1 = 80
EP1 = NW * CPW1 * CH
ACC1 = 10240
PAD1 = EP1 - EDGES

CPW2 = 128
EP2 = NS * CPW2 * CH
PAD2 = EP2 - MPE
ACC2 = 8192
FROWS = 5120
FCH = FROWS // CH

SEG1 = 2
SEG2 = 2


def _zero_vmem_2d(ref, nrows):
    def row(r, _):
        for j in range(D // 16):
            ref[r, pl.ds(16 * j, 16)] = jnp.zeros((16,), jnp.float32)
        return 0
    lax.fori_loop(0, nrows, row, 0)


def _fill_vmem_1d(ref, n, val):
    for j in range(n // 16):
        ref[pl.ds(16 * j, 16)] = jnp.full((16,), val, jnp.float32)


def _sc_mesh():
    return plsc.VectorSubcoreMesh(core_axis_name="c", subcore_axis_name="s")


def _sc1_body(src_hbm, dst_hbm, table_hbm, acc_out, cnt_out0, cnt_out1,
              idx_s, idx_d, rows0, rows1, ones, zvec, acc_sh, cnt_sh,
              gs0, gs1, ss0, ss1, cs0):
    c = lax.axis_index("c")
    s = lax.axis_index("s")
    w = c * NS + s

    _zero_vmem_2d(rows0, CH)
    _fill_vmem_1d(zvec, ACC1 // NS, 0.0)
    _fill_vmem_1d(ones, CH, 1.0)

    rpt = ACC1 // NS
    for k in range(rpt // CH):
        pltpu.sync_copy(rows0, acc_sh.at[pl.ds(s * rpt + k * CH, CH)])
    pltpu.sync_copy(zvec, cnt_sh.at[pl.ds(s * rpt, rpt)])
    plsc.subcore_barrier()

    def drain_s(rbuf, sem_s):
        pltpu.make_async_copy(rbuf, acc_sh.at[idx_d.at[0]], sem_s).wait()

    seg = CPW1 // SEG1
    for sg in range(SEG1):
        base = w * CPW1 + sg * seg
        pltpu.sync_copy(src_hbm.at[pl.ds(base, seg)], idx_s)
        pltpu.sync_copy(dst_hbm.at[pl.ds(base, seg)], idx_d)

        def rnd(r, _):
            @pl.when(r > 0)
            def _():
                drain_s(rows0, ss0)
                drain_s(rows1, ss1)
            g0 = pltpu.async_copy(table_hbm.at[idx_s.at[2 * r]], rows0, gs0)
            g1 = pltpu.async_copy(table_hbm.at[idx_s.at[2 * r + 1]], rows1,
                                  gs1)
            g0.wait()
            pltpu.async_copy(rows0, acc_sh.at[idx_d.at[2 * r]], ss0,
                             add=True)
            g1.wait()
            pltpu.async_copy(rows1, acc_sh.at[idx_d.at[2 * r + 1]], ss1,
                             add=True)
            return 0
        lax.fori_loop(0, seg // 2, rnd, 0)
        drain_s(rows0, ss0)
        drain_s(rows1, ss1)

        cds = [pltpu.async_copy(ones, cnt_sh.at[idx_d.at[j]], cs0, add=True)
               for j in range(seg)]
        for dsc in cds:
            dsc.wait()

    plsc.subcore_barrier()
    pltpu.sync_copy(acc_sh.at[pl.ds(s * rpt, rpt)],
                    acc_out.at[c, pl.ds(s * rpt, rpt)])

    pltpu.sync_copy(cnt_sh.at[pl.ds(s * rpt, rpt)], zvec)

    @pl.when(c == 0)
    def _():
        pltpu.sync_copy(zvec, cnt_out0.at[pl.ds(s * rpt, rpt)])

    @pl.when(c == 1)
    def _():
        pltpu.sync_copy(zvec, cnt_out1.at[pl.ds(s * rpt, rpt)])


def _run_sc1(src2d, dst2d, table):
    f = pl.kernel(
        _sc1_body,
        out_type=(jax.ShapeDtypeStruct((NC, ACC1, D), jnp.float32),
                  jax.ShapeDtypeStruct((ACC1,), jnp.float32),
                  jax.ShapeDtypeStruct((ACC1,), jnp.float32)),
        mesh=_sc_mesh(),
        scratch_types=[
            pltpu.VMEM((CPW1 // SEG1, CH), jnp.int32),
            pltpu.VMEM((CPW1 // SEG1, CH), jnp.int32),
            pltpu.VMEM((CH, D), jnp.float32),
            pltpu.VMEM((CH, D), jnp.float32),
            pltpu.VMEM((CH,), jnp.float32),
            pltpu.VMEM((ACC1 // NS,), jnp.float32),
            pltpu.VMEM_SHARED((ACC1, D), jnp.float32),
            pltpu.VMEM_SHARED((ACC1,), jnp.float32),
            pltpu.SemaphoreType.DMA,
            pltpu.SemaphoreType.DMA,
            pltpu.SemaphoreType.DMA,
            pltpu.SemaphoreType.DMA,
            pltpu.SemaphoreType.DMA,
        ],
    )
    return f(src2d, dst2d, table)


def _sc2_body(e_hbm, rw_hbm, src0, dst0, src1, dst1, eids0, eids1,
              fidx0, fidx1, hidx, ridx,
              mp_acc, mp_cnt0, mp_cnt1, feat, hg, rg,
              idx_s, idx_d, rows0, rows1, ones, zvec, cidx,
              fidx_v, hridx_v, acc_sh, cnt_sh, eids_sh,
              es0, gs0, gs1, ss0, ss1, cs0):
    c = lax.axis_index("c")
    s = lax.axis_index("s")

    _zero_vmem_2d(rows0, CH)
    _fill_vmem_1d(zvec, ACC2 // NS, 0.0)
    _fill_vmem_1d(ones, CH, 1.0)

    rpt = ACC2 // NS
    for k in range(rpt // CH):
        pltpu.sync_copy(rows0, acc_sh.at[pl.ds(s * rpt + k * CH, CH)])
    pltpu.sync_copy(zvec, cnt_sh.at[pl.ds(s * rpt, rpt)])

    @pl.when((c == 0) & (s == 0))
    def _():
        pltpu.sync_copy(eids0, eids_sh)

    @pl.when((c == 1) & (s == 0))
    def _():
        pltpu.sync_copy(eids1, eids_sh)

    plsc.subcore_barrier()

    def drain_s(rbuf, sem_s):
        pltpu.make_async_copy(rbuf, acc_sh.at[idx_d.at[0]], sem_s).wait()

    def graph(src_hbm, dst_hbm):
        seg = CPW2 // SEG2
        for sg in range(SEG2):
            base = s * CPW2 + sg * seg
            pltpu.sync_copy(src_hbm.at[pl.ds(base, seg)], idx_s)
            pltpu.sync_copy(dst_hbm.at[pl.ds(base, seg)], idx_d)

            comp = [pltpu.async_copy(eids_sh.at[idx_s.at[j]], cidx.at[j],
                                     es0) for j in range(seg)]
            for dsc in comp:
                dsc.wait()

            def rnd(r, _):
                @pl.when(r > 0)
                def _():
                    drain_s(rows0, ss0)
                    drain_s(rows1, ss1)
                g0 = pltpu.async_copy(e_hbm.at[cidx.at[2 * r]], rows0, gs0)
                g1 = pltpu.async_copy(e_hbm.at[cidx.at[2 * r + 1]], rows1,
                                      gs1)
                g0.wait()
                pltpu.async_copy(rows0, acc_sh.at[idx_d.at[2 * r]], ss0,
                                 add=True)
                g1.wait()
                pltpu.async_copy(rows1, acc_sh.at[idx_d.at[2 * r + 1]], ss1,
                                 add=True)
                return 0
            lax.fori_loop(0, seg // 2, rnd, 0)
            drain_s(rows0, ss0)
            drain_s(rows1, ss1)

            cds = [pltpu.async_copy(ones, cnt_sh.at[idx_d.at[j]], cs0,
                                    add=True) for j in range(seg)]
            for dsc in cds:
                dsc.wait()

    @pl.when(c == 0)
    def _():
        graph(src0, dst0)

    @pl.when(c == 1)
    def _():
        graph(src1, dst1)

    @pl.when(c == 0)
    def _():
        pltpu.sync_copy(fidx0, fidx_v)

    @pl.when(c == 1)
    def _():
        pltpu.sync_copy(fidx1, fidx_v)

    for k in range(-(-FCH // NS)):
        ch = s + k * NS

        @pl.when(ch < FCH)
        def _():
            pltpu.sync_copy(e_hbm.at[fidx_v.at[ch]], rows0)
            pltpu.sync_copy(rows0, feat.at[c, pl.ds(ch * CH, CH)])

    @pl.when((c == 0) & (s < B // CH))
    def _():
        pltpu.sync_copy(hidx, hridx_v)
        pltpu.sync_copy(e_hbm.at[hridx_v.at[s]], rows1)
        pltpu.sync_copy(rows1, hg.at[pl.ds(s * CH, CH)])

    @pl.when((c == 1) & (s < B // CH))
    def _():
        pltpu.sync_copy(ridx, hridx_v)
        pltpu.sync_copy(rw_hbm.at[hridx_v.at[s]], rows1)
        pltpu.sync_copy(rows1, rg.at[pl.ds(s * CH, CH)])

    plsc.subcore_barrier()
    opt = FROWS // NS
    pltpu.sync_copy(acc_sh.at[pl.ds(s * opt, opt)],
                    mp_acc.at[c, pl.ds(s * opt, opt)])

    pltpu.sync_copy(cnt_sh.at[pl.ds(s * opt, opt)], zvec.at[pl.ds(0, opt)])

    @pl.when(c == 0)
    def _():
        pltpu.sync_copy(zvec.at[pl.ds(0, opt)], mp_cnt0.at[pl.ds(s * opt, opt)])

    @pl.when(c == 1)
    def _():
        pltpu.sync_copy(zvec.at[pl.ds(0, opt)], mp_cnt1.at[pl.ds(s * opt, opt)])


def _run_sc2(e, rw, src0, dst0, src1, dst1, eids0, eids1, fidx0, fidx1,
             hidx, ridx):
    f = pl.kernel(
        _sc2_body,
        out_type=(jax.ShapeDtypeStruct((NC, FROWS, D), jnp.float32),
                  jax.ShapeDtypeStruct((FROWS,), jnp.float32),
                  jax.ShapeDtypeStruct((FROWS,), jnp.float32),
                  jax.ShapeDtypeStruct((NC, FROWS, D), jnp.float32),
                  jax.ShapeDtypeStruct((B, D), jnp.float32),
                  jax.ShapeDtypeStruct((B, D), jnp.float32)),
        mesh=_sc_mesh(),
        scratch_types=[
            pltpu.VMEM((CPW2 // SEG2, CH), jnp.int32),
            pltpu.VMEM((CPW2 // SEG2, CH), jnp.int32),
            pltpu.VMEM((CH, D), jnp.float32),
            pltpu.VMEM((CH, D), jnp.float32),
            pltpu.VMEM((CH,), jnp.float32),
            pltpu.VMEM((ACC2 // NS,), jnp.float32),
            pltpu.VMEM((CPW2 // SEG2, CH), jnp.int32),
            pltpu.VMEM((FCH, CH), jnp.int32),
            pltpu.VMEM((B // CH, CH), jnp.int32),
            pltpu.VMEM_SHARED((ACC2, D), jnp.float32),
            pltpu.VMEM_SHARED((ACC2,), jnp.float32),
            pltpu.VMEM_SHARED((MPN,), jnp.int32),
            pltpu.SemaphoreType.DMA,
            pltpu.SemaphoreType.DMA,
            pltpu.SemaphoreType.DMA,
            pltpu.SemaphoreType.DMA,
            pltpu.SemaphoreType.DMA,
            pltpu.SemaphoreType.DMA,
        ],
    )
    return f(e, rw, src0, dst0, src1, dst1, eids0, eids1, fidx0, fidx1,
             hidx, ridx)


def _tca_body(acc_ref, cnt0_ref, cnt1_ref, ew_ref, w_ref, root_ref, b_ref,
              e_ref):
    acc = acc_ref[0] + acc_ref[1]
    cnt = jnp.maximum(cnt0_ref[...] + cnt1_ref[...], 1.0)
    agg = acc / cnt
    e_ref[...] = jnp.tanh(
        jnp.dot(agg, w_ref[...], preferred_element_type=jnp.float32)
        + jnp.dot(ew_ref[...], root_ref[...], preferred_element_type=jnp.float32)
        + b_ref[...])


def _run_tca(acc_p, cnt0, cnt1, ew, w, root, b):
    blk = 1000
    grid = N // blk
    return pl.pallas_call(
        _tca_body,
        grid=(grid,),
        in_specs=[
            pl.BlockSpec((NC, blk, D), lambda i: (0, i, 0)),
            pl.BlockSpec((blk, 1), lambda i: (i, 0)),
            pl.BlockSpec((blk, 1), lambda i: (i, 0)),
            pl.BlockSpec((blk, D), lambda i: (i, 0)),
            pl.BlockSpec((D, D), lambda i: (0, 0)),
            pl.BlockSpec((D, D), lambda i: (0, 0)),
            pl.BlockSpec((1, D), lambda i: (0, 0)),
        ],
        out_specs=pl.BlockSpec((blk, D), lambda i: (i, 0)),
        out_shape=jax.ShapeDtypeStruct((N, D), jnp.float32),
    )(acc_p, cnt0.reshape(ACC1, 1), cnt1.reshape(ACC1, 1), ew, w, root,
      b.reshape(1, D))


def _tcb_body(h_ref, r_ref, e_ref, out_ref):
    x = h_ref[...] * r_ref[...]
    out_ref[...] = lax.dot_general(
        x, e_ref[...], (((1,), (1,)), ((), ())),
        preferred_element_type=jnp.float32)


def _run_tcb(hg, rg, e):
    blk = 128
    grid = B // blk
    return pl.pallas_call(
        _tcb_body,
        grid=(grid,),
        in_specs=[
            pl.BlockSpec((blk, D), lambda i: (i, 0)),
            pl.BlockSpec((blk, D), lambda i: (i, 0)),
            pl.BlockSpec((N, D), lambda i: (0, 0)),
        ],
        out_specs=pl.BlockSpec((blk, N), lambda i: (i, 0)),
        out_shape=jax.ShapeDtypeStruct((B, N), jnp.float32),
    )(hg, rg, e)


def _tcc_body(acc_ref, cnt0_ref, cnt1_ref, feat_ref, eh_ref,
              w0_ref, r0_ref, b0_ref, w1_ref, r1_ref, b1_ref,
              sw1_ref, sb1_ref, sw2_ref, pw_ref, pb_ref, out_ref):
    gw = (w0_ref, w1_ref)
    gr = (r0_ref, r1_ref)
    gb = (b0_ref, b1_ref)
    gc = (cnt0_ref, cnt1_ref)
    sems = []
    ws = []
    for g in range(2):
        cnt = jnp.maximum(gc[g][:NREG], 1.0)
        agg = acc_ref[g, :NREG] / cnt
        sg = jnp.dot(agg, gw[g][...], preferred_element_type=jnp.float32)
        sg = sg + jnp.dot(feat_ref[g, :NREG], gr[g][...],
                          preferred_element_type=jnp.float32)
        sg = jnp.maximum(sg + gb[g][...], 0.0)
        t = jnp.tanh(jnp.dot(sg, sw1_ref[...],
                             preferred_element_type=jnp.float32) + sb1_ref[...])
        ws.append(jnp.mean(jnp.sum(t * sw2_ref[...], axis=1)))
        sems.append(sg)
    m = jnp.maximum(ws[0], ws[1])
    e0 = jnp.exp(ws[0] - m)
    e1 = jnp.exp(ws[1] - m)
    h_out = (e0 * sems[0] + e1 * sems[1]) / (e0 + e1)
    out_ref[...] = (jnp.dot(h_out, pw_ref[...], preferred_element_type=jnp.float32)
                    + pb_ref[...] + eh_ref[...])


def _run_tcc(mp_acc, mp_cnt0, mp_cnt1, feat, eh, w0, r0, b0, w1, r1, b1,
             sw1, sb1, sw2, pw, pb):
    return pl.pallas_call(
        _tcc_body,
        out_shape=jax.ShapeDtypeStruct((NREG, D), jnp.float32),
    )(mp_acc, mp_cnt0.reshape(FROWS, 1), mp_cnt1.reshape(FROWS, 1), feat,
      eh, w0, r0, b0.reshape(1, D), w1, r1, b1.reshape(1, D), sw1,
      sb1.reshape(1, D), sw2.reshape(1, D), pw, pb.reshape(1, D))


def kernel(E_weight, R_weight, rgcn_W, rgcn_root, rgcn_b,
           gnn0_W, gnn0_root, gnn0_b, gnn1_W, gnn1_root, gnn1_b,
           sem_W1, sem_b1, sem_W2, pred_W, pred_b,
           h_idx, r_idx, edge_index, g0_edge_index, g0_eids,
           g1_edge_index, g1_eids):
    h_idx = h_idx.astype(jnp.int32)
    r_idx = r_idx.astype(jnp.int32)

    pad_src = (jnp.arange(PAD1, dtype=jnp.int32) * 37) % N
    pad_dst = N + (jnp.arange(PAD1, dtype=jnp.int32) % (ACC1 - N))
    src2d = jnp.concatenate([edge_index[0], pad_src]).reshape(EP1 // CH, CH)
    dst2d = jnp.concatenate([edge_index[1], pad_dst]).reshape(EP1 // CH, CH)

    acc_p, cnt0, cnt1 = _run_sc1(src2d, dst2d, E_weight)
    e = _run_tca(acc_p, cnt0, cnt1, E_weight, rgcn_W, rgcn_root, rgcn_b)

    mp_pad_src = (jnp.arange(PAD2, dtype=jnp.int32) * 29) % MPN
    mp_pad_dst = MPN + (jnp.arange(PAD2, dtype=jnp.int32) % (ACC2 - MPN))
    s0 = jnp.concatenate([g0_edge_index[0], mp_pad_src]).reshape(EP2 // CH, CH)
    d0 = jnp.concatenate([g0_edge_index[1], mp_pad_dst]).reshape(EP2 // CH, CH)
    s1 = jnp.concatenate([g1_edge_index[0], mp_pad_src]).reshape(EP2 // CH, CH)
    d1 = jnp.concatenate([g1_edge_index[1], mp_pad_dst]).reshape(EP2 // CH, CH)
    fpad = jnp.zeros((FROWS - NREG,), jnp.int32)
    fidx0 = jnp.concatenate([g0_eids[:NREG], fpad]).reshape(FCH, CH)
    fidx1 = jnp.concatenate([g1_eids[:NREG], fpad]).reshape(FCH, CH)
    hidx = h_idx.reshape(B // CH, CH)
    ridx = r_idx.reshape(B // CH, CH)

    mp_acc, mp_cnt0, mp_cnt1, feat, hg, rg = _run_sc2(
        e, R_weight, s0, d0, s1, d1, g0_eids, g1_eids, fidx0, fidx1,
        hidx, ridx)

    pred = _run_tcb(hg, rg, e)
    e_reg = _run_tcc(mp_acc, mp_cnt0, mp_cnt1, feat, e[:NREG], gnn0_W,
                     gnn0_root, gnn0_b, gnn1_W, gnn1_root, gnn1_b, sem_W1,
                     sem_b1, sem_W2, pred_W, pred_b)
    return (e_reg, pred)


# speedup vs baseline: 1.0666x; 1.0045x over previous
import functools

import jax
import jax.numpy as jnp
import numpy as np
from jax import lax
from jax.experimental import pallas as pl
from jax.experimental.pallas import tpu as pltpu
from jax.experimental.pallas import tpu_sc as plsc

N = 10000
D = 128
NR = 16
NREG = 5000
B = 1024
EDGES = 320000
MPN = 8000
MPE = 256000

NC = 2
NS = 16
NW = NC * NS
CH = 128

CPW1 = 80
EP1 = NW * CPW1 * CH
ACC1 = 10240
PAD1 = EP1 - EDGES

CPW2 = 128
EP2 = NS * CPW2 * CH
PAD2 = EP2 - MPE
ACC2 = 8192
FROWS = 5120
FCH = FROWS // CH

SEG1 = 2
SEG2 = 2


def _zero_vmem_2d(ref, nrows):
    def row(r, _):
        for j in range(D // 16):
            ref[r, pl.ds(16 * j, 16)] = jnp.zeros((16,), jnp.float32)
        return 0
    lax.fori_loop(0, nrows, row, 0)


def _fill_vmem_1d(ref, n, val):
    for j in range(n // 16):
        ref[pl.ds(16 * j, 16)] = jnp.full((16,), val, jnp.float32)


def _sc_mesh():
    return plsc.VectorSubcoreMesh(core_axis_name="c", subcore_axis_name="s")


def _sc1_body(src_hbm, dst_hbm, table_hbm, acc_out, cnt_out0, cnt_out1,
              idx_s, idx_d, rows0, rows1, ones, zvec, acc_sh, cnt_sh,
              gs0, gs1, ss0, ss1, cs0):
    c = lax.axis_index("c")
    s = lax.axis_index("s")
    w = c * NS + s

    _zero_vmem_2d(rows0, CH)
    _fill_vmem_1d(zvec, ACC1 // NS, 0.0)
    _fill_vmem_1d(ones, CH, 1.0)

    rpt = ACC1 // NS
    for k in range(rpt // CH):
        pltpu.sync_copy(rows0, acc_sh.at[pl.ds(s * rpt + k * CH, CH)])
    pltpu.sync_copy(zvec, cnt_sh.at[pl.ds(s * rpt, rpt)])
    plsc.subcore_barrier()

    def drain_s(rbuf, sem_s):
        pltpu.make_async_copy(rbuf, acc_sh.at[idx_d.at[0]], sem_s).wait()

    seg = CPW1 // SEG1
    for sg in range(SEG1):
        base = w * CPW1 + sg * seg
        pltpu.sync_copy(src_hbm.at[pl.ds(base, seg)], idx_s)
        pltpu.sync_copy(dst_hbm.at[pl.ds(base, seg)], idx_d)

        def rnd(r, _):
            @pl.when(r > 0)
            def _():
                drain_s(rows0, ss0)
                drain_s(rows1, ss1)
            g0 = pltpu.async_copy(table_hbm.at[idx_s.at[2 * r]], rows0, gs0)
            g1 = pltpu.async_copy(table_hbm.at[idx_s.at[2 * r + 1]], rows1,
                                  gs1)
            g0.wait()
            pltpu.async_copy(rows0, acc_sh.at[idx_d.at[2 * r]], ss0,
                             add=True)
            g1.wait()
            pltpu.async_copy(rows1, acc_sh.at[idx_d.at[2 * r + 1]], ss1,
                             add=True)
            return 0
        lax.fori_loop(0, seg // 2, rnd, 0)
        drain_s(rows0, ss0)
        drain_s(rows1, ss1)

        cds = [pltpu.async_copy(ones, cnt_sh.at[idx_d.at[j]], cs0, add=True)
               for j in range(seg)]
        for dsc in cds:
            dsc.wait()

    plsc.subcore_barrier()
    pltpu.sync_copy(acc_sh.at[pl.ds(s * rpt, rpt)],
                    acc_out.at[c, pl.ds(s * rpt, rpt)])

    pltpu.sync_copy(cnt_sh.at[pl.ds(s * rpt, rpt)], zvec)

    @pl.when(c == 0)
    def _():
        pltpu.sync_copy(zvec, cnt_out0.at[pl.ds(s * rpt, rpt)])

    @pl.when(c == 1)
    def _():
        pltpu.sync_copy(zvec, cnt_out1.at[pl.ds(s * rpt, rpt)])


def _run_sc1(src2d, dst2d, table):
    f = pl.kernel(
        _sc1_body,
        out_type=(jax.ShapeDtypeStruct((NC, ACC1, D), jnp.float32),
                  jax.ShapeDtypeStruct((ACC1,), jnp.float32),
                  jax.ShapeDtypeStruct((ACC1,), jnp.float32)),
        mesh=_sc_mesh(),
        scratch_types=[
            pltpu.VMEM((CPW1 // SEG1, CH), jnp.int32),
            pltpu.VMEM((CPW1 // SEG1, CH), jnp.int32),
            pltpu.VMEM((CH, D), jnp.float32),
            pltpu.VMEM((CH, D), jnp.float32),
            pltpu.VMEM((CH,), jnp.float32),
            pltpu.VMEM((ACC1 // NS,), jnp.float32),
            pltpu.VMEM_SHARED((ACC1, D), jnp.float32),
            pltpu.VMEM_SHARED((ACC1,), jnp.float32),
            pltpu.SemaphoreType.DMA,
            pltpu.SemaphoreType.DMA,
            pltpu.SemaphoreType.DMA,
            pltpu.SemaphoreType.DMA,
            pltpu.SemaphoreType.DMA,
        ],
    )
    return f(src2d, dst2d, table)


def _sc2_body(e_hbm, rw_hbm, src0, dst0, src1, dst1, eids0, eids1,
              fidx0, fidx1, hidx, ridx,
              mp_acc, mp_cnt0, mp_cnt1, feat, hg, rg,
              idx_s, idx_d, rows0, rows1, ones, zvec, cidx,
              fidx_v, hridx_v, acc_sh, cnt_sh, eids_sh,
              es0, gs0, gs1, ss0, ss1, cs0):
    c = lax.axis_index("c")
    s = lax.axis_index("s")

    _zero_vmem_2d(rows0, CH)
    _fill_vmem_1d(zvec, ACC2 // NS, 0.0)
    _fill_vmem_1d(ones, CH, 1.0)

    rpt = ACC2 // NS
    for k in range(rpt // CH):
        pltpu.sync_copy(rows0, acc_sh.at[pl.ds(s * rpt + k * CH, CH)])
    pltpu.sync_copy(zvec, cnt_sh.at[pl.ds(s * rpt, rpt)])

    @pl.when((c == 0) & (s == 0))
    def _():
        pltpu.sync_copy(eids0, eids_sh)

    @pl.when((c == 1) & (s == 0))
    def _():
        pltpu.sync_copy(eids1, eids_sh)

    plsc.subcore_barrier()

    def drain_s(rbuf, sem_s):
        pltpu.make_async_copy(rbuf, acc_sh.at[idx_d.at[0]], sem_s).wait()

    def graph(src_hbm, dst_hbm):
        seg = CPW2 // SEG2
        for sg in range(SEG2):
            base = s * CPW2 + sg * seg
            pltpu.sync_copy(src_hbm.at[pl.ds(base, seg)], idx_s)
            pltpu.sync_copy(dst_hbm.at[pl.ds(base, seg)], idx_d)

            comp = [pltpu.async_copy(eids_sh.at[idx_s.at[j]], cidx.at[j],
                                     es0) for j in range(seg)]
            for dsc in comp:
                dsc.wait()

            def rnd(r, _):
                @pl.when(r > 0)
                def _():
                    drain_s(rows0, ss0)
                    drain_s(rows1, ss1)
                g0 = pltpu.async_copy(e_hbm.at[cidx.at[2 * r]], rows0, gs0)
                g1 = pltpu.async_copy(e_hbm.at[cidx.at[2 * r + 1]], rows1,
                                      gs1)
                g0.wait()
                pltpu.async_copy(rows0, acc_sh.at[idx_d.at[2 * r]], ss0,
                                 add=True)
                g1.wait()
                pltpu.async_copy(rows1, acc_sh.at[idx_d.at[2 * r + 1]], ss1,
                                 add=True)
                return 0
            lax.fori_loop(0, seg // 2, rnd, 0)
            drain_s(rows0, ss0)
            drain_s(rows1, ss1)

            cds = [pltpu.async_copy(ones, cnt_sh.at[idx_d.at[j]], cs0,
                                    add=True) for j in range(seg)]
            for dsc in cds:
                dsc.wait()

    @pl.when(c == 0)
    def _():
        graph(src0, dst0)

    @pl.when(c == 1)
    def _():
        graph(src1, dst1)

    @pl.when(c == 0)
    def _():
        pltpu.sync_copy(fidx0, fidx_v)

    @pl.when(c == 1)
    def _():
        pltpu.sync_copy(fidx1, fidx_v)

    g0 = pltpu.async_copy(e_hbm.at[fidx_v.at[s]], rows0, gs0)
    g1 = pltpu.async_copy(e_hbm.at[fidx_v.at[s + NS]], rows1, gs1)
    g0.wait()
    w0 = pltpu.async_copy(rows0, feat.at[c, pl.ds(s * CH, CH)], ss0)
    g1.wait()
    w1 = pltpu.async_copy(rows1, feat.at[c, pl.ds((s + NS) * CH, CH)], ss1)
    w0.wait()

    @pl.when(s < FCH - 2 * NS)
    def _():
        pltpu.async_copy(e_hbm.at[fidx_v.at[s + 2 * NS]], rows0, gs0).wait()
        pltpu.async_copy(rows0, feat.at[c, pl.ds((s + 2 * NS) * CH, CH)],
                         ss0).wait()

    w1.wait()

    @pl.when((c == 0) & (s >= NS - B // CH))
    def _():
        pltpu.sync_copy(hidx, hridx_v)
        hch = s - (NS - B // CH)
        pltpu.async_copy(e_hbm.at[hridx_v.at[hch]], rows1, gs1).wait()
        pltpu.async_copy(rows1, hg.at[pl.ds(hch * CH, CH)], ss1).wait()

    @pl.when((c == 1) & (s >= NS - B // CH))
    def _():
        pltpu.sync_copy(ridx, hridx_v)
        hch = s - (NS - B // CH)
        pltpu.async_copy(rw_hbm.at[hridx_v.at[hch]], rows1, gs1).wait()
        pltpu.async_copy(rows1, rg.at[pl.ds(hch * CH, CH)], ss1).wait()

    plsc.subcore_barrier()
    opt = FROWS // NS
    pltpu.sync_copy(acc_sh.at[pl.ds(s * opt, opt)],
                    mp_acc.at[c, pl.ds(s * opt, opt)])

    pltpu.sync_copy(cnt_sh.at[pl.ds(s * opt, opt)], zvec.at[pl.ds(0, opt)])

    @pl.when(c == 0)
    def _():
        pltpu.sync_copy(zvec.at[pl.ds(0, opt)], mp_cnt0.at[pl.ds(s * opt, opt)])

    @pl.when(c == 1)
    def _():
        pltpu.sync_copy(zvec.at[pl.ds(0, opt)], mp_cnt1.at[pl.ds(s * opt, opt)])


def _run_sc2(e, rw, src0, dst0, src1, dst1, eids0, eids1, fidx0, fidx1,
             hidx, ridx):
    f = pl.kernel(
        _sc2_body,
        out_type=(jax.ShapeDtypeStruct((NC, FROWS, D), jnp.float32),
                  jax.ShapeDtypeStruct((FROWS,), jnp.float32),
                  jax.ShapeDtypeStruct((FROWS,), jnp.float32),
                  jax.ShapeDtypeStruct((NC, FROWS, D), jnp.float32),
                  jax.ShapeDtypeStruct((B, D), jnp.float32),
                  jax.ShapeDtypeStruct((B, D), jnp.float32)),
        mesh=_sc_mesh(),
        scratch_types=[
            pltpu.VMEM((CPW2 // SEG2, CH), jnp.int32),
            pltpu.VMEM((CPW2 // SEG2, CH), jnp.int32),
            pltpu.VMEM((CH, D), jnp.float32),
            pltpu.VMEM((CH, D), jnp.float32),
            pltpu.VMEM((CH,), jnp.float32),
            pltpu.VMEM((ACC2 // NS,), jnp.float32),
            pltpu.VMEM((CPW2 // SEG2, CH), jnp.int32),
            pltpu.VMEM((FCH, CH), jnp.int32),
            pltpu.VMEM((B // CH, CH), jnp.int32),
            pltpu.VMEM_SHARED((ACC2, D), jnp.float32),
            pltpu.VMEM_SHARED((ACC2,), jnp.float32),
            pltpu.VMEM_SHARED((MPN,), jnp.int32),
            pltpu.SemaphoreType.DMA,
            pltpu.SemaphoreType.DMA,
            pltpu.SemaphoreType.DMA,
            pltpu.SemaphoreType.DMA,
            pltpu.SemaphoreType.DMA,
            pltpu.SemaphoreType.DMA,
        ],
    )
    return f(e, rw, src0, dst0, src1, dst1, eids0, eids1, fidx0, fidx1,
             hidx, ridx)


def _tca_body(acc_ref, cnt0_ref, cnt1_ref, ew_ref, w_ref, root_ref, b_ref,
              e_ref):
    acc = acc_ref[0] + acc_ref[1]
    cnt = jnp.maximum(cnt0_ref[...] + cnt1_ref[...], 1.0)
    agg = acc / cnt
    e_ref[...] = jnp.tanh(
        jnp.dot(agg, w_ref[...], preferred_element_type=jnp.float32)
        + jnp.dot(ew_ref[...], root_ref[...], preferred_element_type=jnp.float32)
        + b_ref[...])


def _run_tca(acc_p, cnt0, cnt1, ew, w, root, b):
    blk = 1000
    grid = N // blk
    return pl.pallas_call(
        _tca_body,
        grid=(grid,),
        in_specs=[
            pl.BlockSpec((NC, blk, D), lambda i: (0, i, 0)),
            pl.BlockSpec((blk, 1), lambda i: (i, 0)),
            pl.BlockSpec((blk, 1), lambda i: (i, 0)),
            pl.BlockSpec((blk, D), lambda i: (i, 0)),
            pl.BlockSpec((D, D), lambda i: (0, 0)),
            pl.BlockSpec((D, D), lambda i: (0, 0)),
            pl.BlockSpec((1, D), lambda i: (0, 0)),
        ],
        out_specs=pl.BlockSpec((blk, D), lambda i: (i, 0)),
        out_shape=jax.ShapeDtypeStruct((N, D), jnp.float32),
    )(acc_p, cnt0.reshape(ACC1, 1), cnt1.reshape(ACC1, 1), ew, w, root,
      b.reshape(1, D))


def _tcb_body(h_ref, r_ref, e_ref, out_ref):
    x = h_ref[...] * r_ref[...]
    out_ref[...] = lax.dot_general(
        x, e_ref[...], (((1,), (1,)), ((), ())),
        preferred_element_type=jnp.float32)


def _run_tcb(hg, rg, e):
    blk = 128
    grid = B // blk
    return pl.pallas_call(
        _tcb_body,
        grid=(grid,),
        in_specs=[
            pl.BlockSpec((blk, D), lambda i: (i, 0)),
            pl.BlockSpec((blk, D), lambda i: (i, 0)),
            pl.BlockSpec((N, D), lambda i: (0, 0)),
        ],
        out_specs=pl.BlockSpec((blk, N), lambda i: (i, 0)),
        out_shape=jax.ShapeDtypeStruct((B, N), jnp.float32),
    )(hg, rg, e)


def _tcc_body(acc_ref, cnt0_ref, cnt1_ref, feat_ref, eh_ref,
              w0_ref, r0_ref, b0_ref, w1_ref, r1_ref, b1_ref,
              sw1_ref, sb1_ref, sw2_ref, pw_ref, pb_ref, out_ref):
    gw = (w0_ref, w1_ref)
    gr = (r0_ref, r1_ref)
    gb = (b0_ref, b1_ref)
    gc = (cnt0_ref, cnt1_ref)
    sems = []
    ws = []
    for g in range(2):
        cnt = jnp.maximum(gc[g][:NREG], 1.0)
        agg = acc_ref[g, :NREG] / cnt
        sg = jnp.dot(agg, gw[g][...], preferred_element_type=jnp.float32)
        sg = sg + jnp.dot(feat_ref[g, :NREG], gr[g][...],
                          preferred_element_type=jnp.float32)
        sg = jnp.maximum(sg + gb[g][...], 0.0)
        t = jnp.tanh(jnp.dot(sg, sw1_ref[...],
                             preferred_element_type=jnp.float32) + sb1_ref[...])
        ws.append(jnp.mean(jnp.sum(t * sw2_ref[...], axis=1)))
        sems.append(sg)
    m = jnp.maximum(ws[0], ws[1])
    e0 = jnp.exp(ws[0] - m)
    e1 = jnp.exp(ws[1] - m)
    h_out = (e0 * sems[0] + e1 * sems[1]) / (e0 + e1)
    out_ref[...] = (jnp.dot(h_out, pw_ref[...], preferred_element_type=jnp.float32)
                    + pb_ref[...] + eh_ref[...])


def _run_tcc(mp_acc, mp_cnt0, mp_cnt1, feat, eh, w0, r0, b0, w1, r1, b1,
             sw1, sb1, sw2, pw, pb):
    return pl.pallas_call(
        _tcc_body,
        out_shape=jax.ShapeDtypeStruct((NREG, D), jnp.float32),
    )(mp_acc, mp_cnt0.reshape(FROWS, 1), mp_cnt1.reshape(FROWS, 1), feat,
      eh, w0, r0, b0.reshape(1, D), w1, r1, b1.reshape(1, D), sw1,
      sb1.reshape(1, D), sw2.reshape(1, D), pw, pb.reshape(1, D))


def kernel(E_weight, R_weight, rgcn_W, rgcn_root, rgcn_b,
           gnn0_W, gnn0_root, gnn0_b, gnn1_W, gnn1_root, gnn1_b,
           sem_W1, sem_b1, sem_W2, pred_W, pred_b,
           h_idx, r_idx, edge_index, g0_edge_index, g0_eids,
           g1_edge_index, g1_eids):
    h_idx = h_idx.astype(jnp.int32)
    r_idx = r_idx.astype(jnp.int32)

    pad_src = (jnp.arange(PAD1, dtype=jnp.int32) * 37) % N
    pad_dst = N + (jnp.arange(PAD1, dtype=jnp.int32) % (ACC1 - N))
    src2d = jnp.concatenate([edge_index[0], pad_src]).reshape(EP1 // CH, CH)
    dst2d = jnp.concatenate([edge_index[1], pad_dst]).reshape(EP1 // CH, CH)

    acc_p, cnt0, cnt1 = _run_sc1(src2d, dst2d, E_weight)
    e = _run_tca(acc_p, cnt0, cnt1, E_weight, rgcn_W, rgcn_root, rgcn_b)

    mp_pad_src = (jnp.arange(PAD2, dtype=jnp.int32) * 29) % MPN
    mp_pad_dst = MPN + (jnp.arange(PAD2, dtype=jnp.int32) % (ACC2 - MPN))
    s0 = jnp.concatenate([g0_edge_index[0], mp_pad_src]).reshape(EP2 // CH, CH)
    d0 = jnp.concatenate([g0_edge_index[1], mp_pad_dst]).reshape(EP2 // CH, CH)
    s1 = jnp.concatenate([g1_edge_index[0], mp_pad_src]).reshape(EP2 // CH, CH)
    d1 = jnp.concatenate([g1_edge_index[1], mp_pad_dst]).reshape(EP2 // CH, CH)
    fpad = jnp.zeros((FROWS - NREG,), jnp.int32)
    fidx0 = jnp.concatenate([g0_eids[:NREG], fpad]).reshape(FCH, CH)
    fidx1 = jnp.concatenate([g1_eids[:NREG], fpad]).reshape(FCH, CH)
    hidx = h_idx.reshape(B // CH, CH)
    ridx = r_idx.reshape(B // CH, CH)

    mp_acc, mp_cnt0, mp_cnt1, feat, hg, rg = _run_sc2(
        e, R_weight, s0, d0, s1, d1, g0_eids, g1_eids, fidx0, fidx1,
        hidx, ridx)

    pred = _run_tcb(hg, rg, e)
    e_reg = _run_tcc(mp_acc, mp_cnt0, mp_cnt1, feat, e[:NREG], gnn0_W,
                     gnn0_root, gnn0_b, gnn1_W, gnn1_root, gnn1_b, sem_W1,
                     sem_b1, sem_W2, pred_W, pred_b)
    return (e_reg, pred)
